# baseline (speedup 1.0000x reference)
import functools

import jax
import jax.numpy as jnp
from jax import lax
from jax.experimental import pallas as pl
from jax.experimental.pallas import tpu as pltpu
from jax.experimental.pallas import tpu_sc as plsc

L = 8192
B = 64

_info = plsc.get_sparse_core_info()
_NC, _NS, _LANES = _info.num_cores, _info.num_subcores, _info.num_lanes
_NW = _NC * _NS
_ROWS_PER_W = B // _NW

_mesh = plsc.VectorSubcoreMesh(core_axis_name="c", subcore_axis_name="s")


@functools.partial(
    pl.kernel,
    mesh=_mesh,
    out_type=jax.ShapeDtypeStruct((B * L,), jnp.float32),
    scratch_types=[
        pltpu.VMEM((L,), jnp.float32),
        pltpu.VMEM((L,), jnp.float32),
        pltpu.VMEM((L,), jnp.int32),
        pltpu.VMEM((L,), jnp.int32),
        pltpu.VMEM((L,), jnp.float32),
        pltpu.VMEM((L,), jnp.float32),
        pltpu.SemaphoreType.DMA,
        pltpu.SemaphoreType.DMA,
        pltpu.SemaphoreType.DMA,
    ],
    compiler_params=pltpu.CompilerParams(needs_layout_passes=False),
)
def _interleave(
    x_hbm, p_hbm, out_hbm, xv0, xv1, pv0, pv1, ov0, ov1, in_sem0, in_sem1, out_sem
):
    wid = lax.axis_index("s") * _NC + lax.axis_index("c")
    base = wid * _ROWS_PER_W
    rows = ((xv0, pv0, ov0, in_sem0), (xv1, pv1, ov1, in_sem1))

    loads = []
    for r, (xv, pv, ov, sem) in enumerate(rows):
        loads.append((
            pltpu.async_copy(p_hbm.at[base + r], pv, sem),
            pltpu.async_copy(x_hbm.at[pl.ds((base + r) * L, L)], xv, sem),
        ))

    stores = []
    for r, (xv, pv, ov, sem) in enumerate(rows):
        for c in loads[r]:
            c.wait()

        @plsc.parallel_loop(0, L, step=_LANES, unroll=8)
        def _gather(j, xv=xv, pv=pv, ov=ov):
            idx = pv[pl.ds(j, _LANES)]
            ov[pl.ds(j, _LANES)] = plsc.load_gather(xv, [idx])

        stores.append(
            pltpu.async_copy(ov, out_hbm.at[pl.ds((base + r) * L, L)], out_sem)
        )

    for s in stores:
        s.wait()


def kernel(x, perms):
    out = _interleave(x.reshape(B * L), perms)
    return out.reshape(B, L, 1)
